# Initial kernel scaffold; baseline (speedup 1.0000x reference)
import jax
import jax.numpy as jnp
from jax.experimental import pallas as pl


def kernel(position_ids, pos_emb):
    raise NotImplementedError("write your pallas kernel here")


# speedup vs baseline: 2.3590x; 2.3590x over previous
import jax
import jax.numpy as jnp
from jax import lax
from jax.experimental import pallas as pl
from jax.experimental.pallas import tpu as pltpu
from jax.experimental.pallas import tpu_sc as plsc

_D = 1024
_NC = 2
_NS = 16
_NW = _NC * _NS
_CH = 32


def _sc_gather(idx_flat, pos_emb, n):
    b_per_w = n // _NW
    n_chunk = b_per_w // _CH
    mesh = plsc.VectorSubcoreMesh(core_axis_name="core", subcore_axis_name="subcore")

    @pl.kernel(
        out_type=jax.ShapeDtypeStruct((n, _D), pos_emb.dtype),
        mesh=mesh,
        scratch_types=[
            pltpu.VMEM((b_per_w,), jnp.int32),
            pltpu.VMEM((_CH, _D), jnp.float32),
            pltpu.VMEM((_CH, _D), jnp.float32),
            pltpu.SemaphoreType.DMA,
            pltpu.SemaphoreType.DMA,
        ],
    )
    def gather_kernel(table_hbm, idx_hbm, out_hbm, idx_v, rows0, rows1, sem0, sem1):
        wid = lax.axis_index("subcore") * _NC + lax.axis_index("core")
        base = wid * b_per_w
        pltpu.sync_copy(idx_hbm.at[pl.ds(base, b_per_w)], idx_v)

        pltpu.async_copy(table_hbm.at[idx_v.at[pl.ds(0, _CH)]], rows0, sem0)

        def wait_gather(rows, sem):
            pltpu.make_async_copy(table_hbm.at[pl.ds(0, _CH)], rows, sem).wait()

        @pl.loop(0, n_chunk, step=2)
        def _(c):
            wait_gather(rows0, sem0)
            pltpu.async_copy(
                table_hbm.at[idx_v.at[pl.ds((c + 1) * _CH, _CH)]], rows1, sem1
            )
            pltpu.sync_copy(rows0, out_hbm.at[pl.ds(base + c * _CH, _CH)])

            @pl.when(c + 2 < n_chunk)
            def _():
                pltpu.async_copy(
                    table_hbm.at[idx_v.at[pl.ds((c + 2) * _CH, _CH)]], rows0, sem0
                )

            wait_gather(rows1, sem1)
            pltpu.sync_copy(rows1, out_hbm.at[pl.ds(base + (c + 1) * _CH, _CH)])

    return gather_kernel(pos_emb, idx_flat)


def kernel(position_ids, pos_emb):
    b, s = position_ids.shape
    n = b * s
    idx_flat = position_ids.reshape(n).astype(jnp.int32)
    out = _sc_gather(idx_flat, pos_emb, n)
    return out.reshape(b, s, _D)


# speedup vs baseline: 2.3610x; 1.0008x over previous
import jax
import jax.numpy as jnp
from jax import lax
from jax.experimental import pallas as pl
from jax.experimental.pallas import tpu as pltpu
from jax.experimental.pallas import tpu_sc as plsc

_D = 1024
_NC = 2
_NS = 16
_NW = _NC * _NS
_CH = 32


def _sc_gather(idx_flat, pos_emb, n):
    b_per_w = n // _NW
    n_chunk = b_per_w // _CH
    mesh = plsc.VectorSubcoreMesh(core_axis_name="core", subcore_axis_name="subcore")

    @pl.kernel(
        out_type=jax.ShapeDtypeStruct((n, _D), pos_emb.dtype),
        mesh=mesh,
        scratch_types=[
            pltpu.VMEM((b_per_w,), jnp.int32),
            pltpu.VMEM((_CH, _D), jnp.float32),
            pltpu.VMEM((_CH, _D), jnp.float32),
            pltpu.SemaphoreType.DMA,
            pltpu.SemaphoreType.DMA,
            pltpu.SemaphoreType.DMA,
            pltpu.SemaphoreType.DMA,
        ],
    )
    def gather_kernel(
        table_hbm, idx_hbm, out_hbm, idx_v, rows0, rows1, gsem0, gsem1, ssem0, ssem1
    ):
        wid = lax.axis_index("subcore") * _NC + lax.axis_index("core")
        base = wid * b_per_w
        pltpu.sync_copy(idx_hbm.at[pl.ds(base, b_per_w)], idx_v)

        pltpu.async_copy(table_hbm.at[idx_v.at[pl.ds(0, _CH)]], rows0, gsem0)

        def wait_gather(rows, sem):
            pltpu.make_async_copy(table_hbm.at[pl.ds(0, _CH)], rows, sem).wait()

        def wait_store(rows, sem):
            pltpu.make_async_copy(rows, out_hbm.at[pl.ds(base, _CH)], sem).wait()

        @pl.loop(0, n_chunk, step=2)
        def _(c):
            wait_gather(rows0, gsem0)

            @pl.when(c > 0)
            def _():
                wait_store(rows1, ssem1)

            pltpu.async_copy(
                table_hbm.at[idx_v.at[pl.ds((c + 1) * _CH, _CH)]], rows1, gsem1
            )
            pltpu.async_copy(rows0, out_hbm.at[pl.ds(base + c * _CH, _CH)], ssem0)
            wait_store(rows0, ssem0)

            @pl.when(c + 2 < n_chunk)
            def _():
                pltpu.async_copy(
                    table_hbm.at[idx_v.at[pl.ds((c + 2) * _CH, _CH)]], rows0, gsem0
                )

            wait_gather(rows1, gsem1)
            pltpu.async_copy(
                rows1, out_hbm.at[pl.ds(base + (c + 1) * _CH, _CH)], ssem1
            )

        wait_store(rows1, ssem1)

    return gather_kernel(pos_emb, idx_flat)


def kernel(position_ids, pos_emb):
    b, s = position_ids.shape
    n = b * s
    idx_flat = position_ids.reshape(n).astype(jnp.int32)
    out = _sc_gather(idx_flat, pos_emb, n)
    return out.reshape(b, s, _D)


# speedup vs baseline: 2.3936x; 1.0138x over previous
import jax
import jax.numpy as jnp
from jax import lax
from jax.experimental import pallas as pl
from jax.experimental.pallas import tpu as pltpu
from jax.experimental.pallas import tpu_sc as plsc

_D = 1024
_NC = 2
_NS = 16
_NW = _NC * _NS
_CH = 32


def _sc_gather(idx_flat, pos_emb, n):
    b_per_w = n // _NW
    n_chunk = b_per_w // _CH
    mesh = plsc.VectorSubcoreMesh(core_axis_name="core", subcore_axis_name="subcore")

    @pl.kernel(
        out_type=jax.ShapeDtypeStruct((n, _D), pos_emb.dtype),
        mesh=mesh,
        scratch_types=[
            pltpu.VMEM((b_per_w,), jnp.int32),
            pltpu.VMEM((_CH, _D), jnp.float32),
            pltpu.VMEM((_CH, _D), jnp.float32),
            pltpu.VMEM((_CH, _D), jnp.float32),
            pltpu.SemaphoreType.DMA,
            pltpu.SemaphoreType.DMA,
            pltpu.SemaphoreType.DMA,
            pltpu.SemaphoreType.DMA,
            pltpu.SemaphoreType.DMA,
            pltpu.SemaphoreType.DMA,
        ],
    )
    def gather_kernel(
        table_hbm, idx_hbm, out_hbm, idx_v,
        rows0, rows1, rows2, gsem0, gsem1, gsem2, ssem0, ssem1, ssem2,
    ):
        wid = lax.axis_index("subcore") * _NC + lax.axis_index("core")
        base = wid * b_per_w
        pltpu.sync_copy(idx_hbm.at[pl.ds(base, b_per_w)], idx_v)

        rows = [rows0, rows1, rows2]
        gsem = [gsem0, gsem1, gsem2]
        ssem = [ssem0, ssem1, ssem2]

        def wait_gather(b):
            pltpu.make_async_copy(table_hbm.at[pl.ds(0, _CH)], rows[b], gsem[b]).wait()

        def wait_store(b):
            pltpu.make_async_copy(rows[b], out_hbm.at[pl.ds(base, _CH)], ssem[b]).wait()

        @pl.loop(0, n_chunk + 3, step=3)
        def _(c):
            for b in range(3):
                k = c + b
                bj = (b + 1) % 3

                @pl.when(jnp.logical_and(k >= 3, k < n_chunk))
                def _():
                    wait_store(b)

                @pl.when(k < n_chunk)
                def _():
                    pltpu.async_copy(
                        table_hbm.at[idx_v.at[pl.ds(k * _CH, _CH)]], rows[b], gsem[b]
                    )

                j = k - 2

                @pl.when(jnp.logical_and(j >= 0, j < n_chunk))
                def _():
                    wait_gather(bj)
                    pltpu.async_copy(
                        rows[bj], out_hbm.at[pl.ds(base + j * _CH, _CH)], ssem[bj]
                    )

        for b in range(3):
            wait_store(b)

    return gather_kernel(pos_emb, idx_flat)


def kernel(position_ids, pos_emb):
    b, s = position_ids.shape
    n = b * s
    idx_flat = position_ids.reshape(n).astype(jnp.int32)
    out = _sc_gather(idx_flat, pos_emb, n)
    return out.reshape(b, s, _D)
